# baseline (speedup 1.0000x reference)
import jax
import jax.numpy as jnp
from jax.experimental import pallas as pl
from jax.experimental.pallas import tpu as pltpu

GAMMA = 0.01
ALPHA = 1.0

ROW_BLOCK = 25000


def _body(x_ref, u_ref, zmu_ref, o_ref):
    o_ref[...] = x_ref[...]

    @pl.when(pl.program_id(0) == 0)
    def _compute():
        k = u_ref.shape[1]
        x = x_ref[:k, :]
        zmu = zmu_ref[...]
        diff = x - zmu
        kern = ALPHA * jnp.exp(-GAMMA * jnp.sum(diff * diff, axis=1,
                                                keepdims=True))
        u = u_ref[...]
        proj = jnp.dot(jnp.dot(diff, u, preferred_element_type=jnp.float32),
                       u.T, preferred_element_type=jnp.float32) + zmu
        o_ref[:k, :] = proj * kern + x * (1.0 - kern)


def _pick_row_block(n, k):
    if n % ROW_BLOCK == 0 and k <= ROW_BLOCK:
        return ROW_BLOCK
    best = n
    for cand in range(max(k, 1), min(n, 32768) + 1):
        if n % cand == 0:
            best = cand
    return best


def kernel(X_all, U, z_mu_local):
    n, d = X_all.shape
    k = U.shape[1]
    blk = _pick_row_block(n, k)
    return pl.pallas_call(
        _body,
        grid=(n // blk,),
        in_specs=[
            pl.BlockSpec((blk, d), lambda i: (i, 0)),
            pl.BlockSpec((d, k), lambda i: (0, 0)),
            pl.BlockSpec((1, d), lambda i: (0, 0)),
        ],
        out_specs=pl.BlockSpec((blk, d), lambda i: (i, 0)),
        out_shape=jax.ShapeDtypeStruct((n, d), X_all.dtype),
        compiler_params=pltpu.CompilerParams(
            dimension_semantics=("parallel",),
        ),
    )(X_all, U, z_mu_local)
